# baseline (speedup 1.0000x reference)
import math

import jax
import jax.numpy as jnp
from jax.experimental import pallas as pl

_BLOCK_ROWS = 512
_HALF_PI = math.pi / 2.0


def _make_gen_block(hidden):
    c = 2.0 * math.log(10000.0) / hidden

    def _gen_block(o_ref):
        rows = o_ref.shape[0]
        pid = pl.program_id(0)
        irow = jax.lax.broadcasted_iota(jnp.int32, (rows, 1), 0)
        pos = (irow + pid * rows).astype(jnp.float32)
        icol = jax.lax.broadcasted_iota(jnp.int32, (1, hidden), 1)
        invfreq = jnp.exp(icol.astype(jnp.float32) * (-c))
        shift = (icol & 1).astype(jnp.float32) * _HALF_PI
        val = jnp.sin(pos * invfreq + shift)
        o_ref[...] = jnp.where(pos == 0.0, 0.0, val)

    return _gen_block


def kernel(x, table):
    seq_len = x.shape[-1]
    hidden = table.shape[1]
    return pl.pallas_call(
        _make_gen_block(hidden),
        grid=(seq_len // _BLOCK_ROWS,),
        out_specs=pl.BlockSpec((_BLOCK_ROWS, hidden), lambda i: (i, 0)),
        out_shape=jax.ShapeDtypeStruct((seq_len, hidden), table.dtype),
    )()


# speedup vs baseline: 3.0426x; 3.0426x over previous
import math

import jax
import jax.numpy as jnp
from jax.experimental import pallas as pl
from jax.experimental.pallas import tpu as pltpu

_BLOCK_ROWS = 512
_SUB = 64
_HALF_PI = math.pi / 2.0


def _make_gen_block(hidden):
    c = 2.0 * math.log(10000.0) / hidden

    def _gen_block(o_ref, sinb_ref, cosb_ref):
        pid = pl.program_id(0)
        icol = jax.lax.broadcasted_iota(jnp.int32, (1, hidden), 1)
        f = jnp.exp(icol.astype(jnp.float32) * (-c))
        phase = (icol & 1).astype(jnp.float32) * _HALF_PI

        @pl.when(pid == 0)
        def _init():
            b = jax.lax.broadcasted_iota(jnp.int32, (_SUB, 1), 0)
            ang = b.astype(jnp.float32) * f + phase
            sinb_ref[...] = jnp.sin(ang)
            cosb_ref[...] = jnp.sin(ang + _HALF_PI)

        sinb = sinb_ref[...]
        cosb = cosb_ref[...]
        for k in range(_BLOCK_ROWS // _SUB):
            a = pid * (_BLOCK_ROWS // _SUB) + k
            anga = (a * _SUB).astype(jnp.float32) * f
            sina = jnp.sin(anga)
            cosa = jnp.sin(anga + _HALF_PI)
            o_ref[pl.ds(k * _SUB, _SUB), :] = sina * cosb + cosa * sinb

        @pl.when(pid == 0)
        def _zero_row0():
            o_ref[0:1, :] = jnp.zeros((1, hidden), jnp.float32)

    return _gen_block


def kernel(x, table):
    seq_len = x.shape[-1]
    hidden = table.shape[1]
    return pl.pallas_call(
        _make_gen_block(hidden),
        grid=(seq_len // _BLOCK_ROWS,),
        out_specs=pl.BlockSpec((_BLOCK_ROWS, hidden), lambda i: (i, 0)),
        out_shape=jax.ShapeDtypeStruct((seq_len, hidden), table.dtype),
        scratch_shapes=[
            pltpu.VMEM((_SUB, hidden), jnp.float32),
            pltpu.VMEM((_SUB, hidden), jnp.float32),
        ],
    )()


# speedup vs baseline: 5.6983x; 1.8728x over previous
import math

import jax
import jax.numpy as jnp
from jax import lax
from jax.experimental import pallas as pl
from jax.experimental.pallas import tpu as pltpu

_BLOCK_ROWS = 512
_SUB = 64
_HALF_PI = math.pi / 2.0


def _make_gen_block(hidden):
    c = 2.0 * math.log(10000.0) / hidden
    chunks = _BLOCK_ROWS // _SUB

    def _gen_block(o_ref, sinb_ref, cosb_ref):
        pid = pl.program_id(0)
        icol = jax.lax.broadcasted_iota(jnp.int32, (1, hidden), 1)
        f = jnp.exp(icol.astype(jnp.float32) * (-c))
        phase = (icol & 1).astype(jnp.float32) * _HALF_PI

        @pl.when(pid == 0)
        def _init():
            b = jax.lax.broadcasted_iota(jnp.int32, (_SUB, 1), 0)
            ang = b.astype(jnp.float32) * f + phase
            sinb_ref[...] = jnp.sin(ang)
            cosb_ref[...] = jnp.sin(ang + _HALF_PI)

        sinb = sinb_ref[...]
        cosb = cosb_ref[...]
        ka = jax.lax.broadcasted_iota(jnp.int32, (chunks, 1), 0) + pid * chunks
        anga = (ka * _SUB).astype(jnp.float32) * f
        sina_blk = jnp.sin(anga)
        cosa_blk = jnp.sin(anga + _HALF_PI)
        for k in range(chunks):
            sina = lax.slice(sina_blk, (k, 0), (k + 1, hidden))
            cosa = lax.slice(cosa_blk, (k, 0), (k + 1, hidden))
            o_ref[pl.ds(k * _SUB, _SUB), :] = sina * cosb + cosa * sinb

        @pl.when(pid == 0)
        def _zero_row0():
            o_ref[0:1, :] = jnp.zeros((1, hidden), jnp.float32)

    return _gen_block


def kernel(x, table):
    seq_len = x.shape[-1]
    hidden = table.shape[1]
    return pl.pallas_call(
        _make_gen_block(hidden),
        grid=(seq_len // _BLOCK_ROWS,),
        out_specs=pl.BlockSpec((_BLOCK_ROWS, hidden), lambda i: (i, 0)),
        out_shape=jax.ShapeDtypeStruct((seq_len, hidden), table.dtype),
        scratch_shapes=[
            pltpu.VMEM((_SUB, hidden), jnp.float32),
            pltpu.VMEM((_SUB, hidden), jnp.float32),
        ],
    )()


# speedup vs baseline: 6.8724x; 1.2060x over previous
import math

import jax
import jax.numpy as jnp
from jax import lax
from jax.experimental import pallas as pl
from jax.experimental.pallas import tpu as pltpu

_BLOCK_ROWS = 512
_SUB = 64
_HALF_PI = math.pi / 2.0


def _make_gen_block(hidden):
    c = 2.0 * math.log(10000.0) / hidden
    chunks = _BLOCK_ROWS // _SUB

    def _gen_block(o_ref, sinb_ref, cosb_ref, sina_ref, cosa_ref, stp_ref):
        pid = pl.program_id(0)
        icol = jax.lax.broadcasted_iota(jnp.int32, (1, hidden), 1)
        f = jnp.exp(icol.astype(jnp.float32) * (-c))

        @pl.when(pid == 0)
        def _init():
            phase = (icol & 1).astype(jnp.float32) * _HALF_PI
            d8 = jax.lax.broadcasted_iota(jnp.int32, (8, 1), 0)
            angd = d8.astype(jnp.float32) * f + phase
            sind = jnp.sin(angd)
            cosd = jnp.sin(angd + _HALF_PI)
            angc = (d8 * 8).astype(jnp.float32) * f
            sinc = jnp.sin(angc)
            cosc = jnp.sin(angc + _HALF_PI)
            for cc in range(8):
                sc = lax.slice(sinc, (cc, 0), (cc + 1, hidden))
                kc = lax.slice(cosc, (cc, 0), (cc + 1, hidden))
                sinb_ref[pl.ds(cc * 8, 8), :] = sc * cosd + kc * sind
                cosb_ref[pl.ds(cc * 8, 8), :] = kc * cosd - sc * sind
            anga = (d8 * _SUB).astype(jnp.float32) * f
            sina_ref[...] = jnp.sin(anga)
            cosa_ref[...] = jnp.sin(anga + _HALF_PI)
            angs = jnp.float32(_BLOCK_ROWS) * f
            stp_ref[0:1, :] = jnp.sin(angs)
            stp_ref[1:2, :] = jnp.sin(angs + _HALF_PI)

        sinb = sinb_ref[...]
        cosb = cosb_ref[...]
        sina_blk = sina_ref[...]
        cosa_blk = cosa_ref[...]
        for k in range(chunks):
            sina = lax.slice(sina_blk, (k, 0), (k + 1, hidden))
            cosa = lax.slice(cosa_blk, (k, 0), (k + 1, hidden))
            o_ref[pl.ds(k * _SUB, _SUB), :] = sina * cosb + cosa * sinb

        @pl.when(pid == 0)
        def _zero_row0():
            o_ref[0:1, :] = jnp.zeros((1, hidden), jnp.float32)

        sstp = stp_ref[0:1, :]
        cstp = stp_ref[1:2, :]
        sina_ref[...] = sina_blk * cstp + cosa_blk * sstp
        cosa_ref[...] = cosa_blk * cstp - sina_blk * sstp

    return _gen_block


def kernel(x, table):
    seq_len = x.shape[-1]
    hidden = table.shape[1]
    return pl.pallas_call(
        _make_gen_block(hidden),
        grid=(seq_len // _BLOCK_ROWS,),
        out_specs=pl.BlockSpec((_BLOCK_ROWS, hidden), lambda i: (i, 0)),
        out_shape=jax.ShapeDtypeStruct((seq_len, hidden), table.dtype),
        scratch_shapes=[
            pltpu.VMEM((_SUB, hidden), jnp.float32),
            pltpu.VMEM((_SUB, hidden), jnp.float32),
            pltpu.VMEM((8, hidden), jnp.float32),
            pltpu.VMEM((8, hidden), jnp.float32),
            pltpu.VMEM((2, hidden), jnp.float32),
        ],
    )()


# speedup vs baseline: 6.8944x; 1.0032x over previous
import math

import jax
import jax.numpy as jnp
from jax import lax
from jax.experimental import pallas as pl
from jax.experimental.pallas import tpu as pltpu

_BLOCK_ROWS = 512
_SUB = 64
_HALF_PI = math.pi / 2.0


def _make_gen_block(hidden):
    c = 2.0 * math.log(10000.0) / hidden
    chunks = _BLOCK_ROWS // _SUB

    def _gen_block(o_ref, sinb_ref, cosb_ref, sina_ref, cosa_ref, stp_ref):
        pid = pl.program_id(0)

        @pl.when(pid == 0)
        def _init():
            icol = jax.lax.broadcasted_iota(jnp.int32, (1, hidden), 1)
            f = jnp.exp(icol.astype(jnp.float32) * (-c))
            phase = (icol & 1).astype(jnp.float32) * _HALF_PI
            d8 = jax.lax.broadcasted_iota(jnp.int32, (8, 1), 0)
            angd = d8.astype(jnp.float32) * f + phase
            sind = jnp.sin(angd)
            cosd = jnp.sin(angd + _HALF_PI)
            angc = (d8 * 8).astype(jnp.float32) * f
            sinc = jnp.sin(angc)
            cosc = jnp.sin(angc + _HALF_PI)
            for cc in range(8):
                sc = lax.slice(sinc, (cc, 0), (cc + 1, hidden))
                kc = lax.slice(cosc, (cc, 0), (cc + 1, hidden))
                sinb_ref[pl.ds(cc * 8, 8), :] = sc * cosd + kc * sind
                cosb_ref[pl.ds(cc * 8, 8), :] = kc * cosd - sc * sind
            anga = (d8 * _SUB).astype(jnp.float32) * f
            sina_ref[...] = jnp.sin(anga)
            cosa_ref[...] = jnp.sin(anga + _HALF_PI)
            angs = jnp.float32(_BLOCK_ROWS) * f
            stp_ref[0:1, :] = jnp.sin(angs)
            stp_ref[1:2, :] = jnp.sin(angs + _HALF_PI)

        sinb = sinb_ref[...]
        cosb = cosb_ref[...]
        sina_blk = sina_ref[...]
        cosa_blk = cosa_ref[...]
        for k in range(chunks):
            sina = lax.slice(sina_blk, (k, 0), (k + 1, hidden))
            cosa = lax.slice(cosa_blk, (k, 0), (k + 1, hidden))
            o_ref[pl.ds(k * _SUB, _SUB), :] = sina * cosb + cosa * sinb

        @pl.when(pid == 0)
        def _zero_row0():
            o_ref[0:1, :] = jnp.zeros((1, hidden), jnp.float32)

        sstp = stp_ref[0:1, :]
        cstp = stp_ref[1:2, :]
        sina_ref[...] = sina_blk * cstp + cosa_blk * sstp
        cosa_ref[...] = cosa_blk * cstp - sina_blk * sstp

    return _gen_block


def kernel(x, table):
    seq_len = x.shape[-1]
    hidden = table.shape[1]
    return pl.pallas_call(
        _make_gen_block(hidden),
        grid=(seq_len // _BLOCK_ROWS,),
        out_specs=pl.BlockSpec((_BLOCK_ROWS, hidden), lambda i: (i, 0)),
        out_shape=jax.ShapeDtypeStruct((seq_len, hidden), table.dtype),
        scratch_shapes=[
            pltpu.VMEM((_SUB, hidden), jnp.float32),
            pltpu.VMEM((_SUB, hidden), jnp.float32),
            pltpu.VMEM((8, hidden), jnp.float32),
            pltpu.VMEM((8, hidden), jnp.float32),
            pltpu.VMEM((2, hidden), jnp.float32),
        ],
    )()
